# baseline (speedup 1.0000x reference)
import functools

import jax
import jax.numpy as jnp
from jax import lax
from jax.experimental import pallas as pl
from jax.experimental.pallas import tpu as pltpu
from jax.experimental.pallas import tpu_sc as plsc

_NUM_CORES = 2
_NUM_SUBCORES = 16
_NUM_WORKERS = _NUM_CORES * _NUM_SUBCORES
_CHUNK = 128


def _gather_kernel(b_per_w, d):
    mesh = plsc.VectorSubcoreMesh(core_axis_name="c", subcore_axis_name="s")
    n_chunks = b_per_w // _CHUNK

    @functools.partial(
        pl.kernel,
        out_type=jax.ShapeDtypeStruct((d, _NUM_WORKERS * b_per_w), jnp.float32),
        mesh=mesh,
        scratch_types=[
            pltpu.VMEM((b_per_w,), jnp.int32),
            pltpu.VMEM((d, b_per_w), jnp.float32),
            pltpu.SemaphoreType.DMA,
        ],
        compiler_params=pltpu.CompilerParams(use_tc_tiling_on_sc=False),
    )
    def k(x_hbm, table_t_hbm, out_t_hbm, idx_v, stage_v, sem):
        wid = lax.axis_index("s") * _NUM_CORES + lax.axis_index("c")
        base = wid * b_per_w
        pltpu.sync_copy(x_hbm.at[pl.ds(base, b_per_w)], idx_v)
        for c in range(n_chunks):
            idx_c = idx_v.at[pl.ds(c * _CHUNK, _CHUNK)]
            for j in range(d):
                pltpu.async_copy(
                    table_t_hbm.at[j].at[idx_c],
                    stage_v.at[j, pl.ds(c * _CHUNK, _CHUNK)],
                    sem,
                )
        pltpu.make_async_copy(
            table_t_hbm.at[:, pl.ds(0, b_per_w)], stage_v, sem
        ).wait()
        pltpu.sync_copy(stage_v, out_t_hbm.at[:, pl.ds(base, b_per_w)])

    return k


@jax.jit
def kernel(x, table):
    (b,) = x.shape
    _, d = table.shape
    b_per_w = b // _NUM_WORKERS
    table_t = jnp.swapaxes(table, 0, 1)
    out_t = _gather_kernel(b_per_w, d)(x, table_t)
    return jnp.swapaxes(out_t, 0, 1)


# speedup vs baseline: 8.2022x; 8.2022x over previous
import functools

import jax
import jax.numpy as jnp
from jax import lax
from jax.experimental import pallas as pl
from jax.experimental.pallas import tpu as pltpu
from jax.experimental.pallas import tpu_sc as plsc

_NUM_CORES = 2
_NUM_SUBCORES = 16
_NUM_WORKERS = _NUM_CORES * _NUM_SUBCORES
_LANES = 16


def _gather_kernel(b_per_w, d):
    mesh = plsc.VectorSubcoreMesh(core_axis_name="c", subcore_axis_name="s")

    @functools.partial(
        pl.kernel,
        out_type=jax.ShapeDtypeStruct((_NUM_WORKERS * b_per_w, d), jnp.float32),
        mesh=mesh,
        scratch_types=[
            pltpu.VMEM((b_per_w,), jnp.int32),
            pltpu.VMEM((b_per_w, d), jnp.float32),
            pltpu.SemaphoreType.DMA,
        ],
    )
    def k(x_hbm, table_hbm, out_hbm, idx_v, rows_v, sem):
        wid = lax.axis_index("s") * _NUM_CORES + lax.axis_index("c")
        base = wid * b_per_w
        pltpu.sync_copy(x_hbm.at[pl.ds(base, b_per_w)], idx_v)

        def issue(g, _):
            v = idx_v[pl.ds(g * _LANES, _LANES)]
            kk = g * _LANES
            for j in range(_LANES):
                pltpu.async_copy(
                    table_hbm.at[pl.ds(v[j], 1), :],
                    rows_v.at[pl.ds(kk + j, 1), :],
                    sem,
                )
            return 0

        lax.fori_loop(0, b_per_w // _LANES, issue, 0)

        pltpu.make_async_copy(
            out_hbm.at[pl.ds(base, b_per_w)], rows_v, sem
        ).wait()
        pltpu.sync_copy(rows_v, out_hbm.at[pl.ds(base, b_per_w)])

    return k


@jax.jit
def kernel(x, table):
    (b,) = x.shape
    _, d = table.shape
    b_per_w = b // _NUM_WORKERS
    return _gather_kernel(b_per_w, d)(x, table)
